# Initial kernel scaffold; baseline (speedup 1.0000x reference)
import jax
import jax.numpy as jnp
from jax.experimental import pallas as pl


def kernel(embedding, W1, b1, W2, b2, Wc, bc, input_nodes, edge_index, output_nodes, labels):
    raise NotImplementedError("write your pallas kernel here")


# speedup vs baseline: 9.0719x; 9.0719x over previous
import functools

import jax
import jax.numpy as jnp
from jax import lax
from jax.experimental import pallas as pl
from jax.experimental.pallas import tpu as pltpu
from jax.experimental.pallas import tpu_sc as plsc

NC = 2
NS = 16
NW = NC * NS
L = 16

N_NODES = 10000
N_PAD = 10240
E_TOT = 320000
D = 128
CH = 128
NCHUNK = E_TOT // CH
STAGE = 79 * CH
B_OUT = 4096
Q = B_OUT // 2
ROWS_PER_TILE = N_PAD // NS

_mesh = plsc.VectorSubcoreMesh(core_axis_name="c", subcore_axis_name="s")
_sc_params = pltpu.CompilerParams(needs_layout_passes=False)


def _zero_zrow(zrow_v):
    zv = jnp.zeros((L,), jnp.float32)

    def zbody(i, carry):
        r = i // (D // L)
        k = i % (D // L)
        zrow_v[r, pl.ds(k * L, L)] = zv
        return carry

    lax.fori_loop(0, CH * (D // L), zbody, 0)


def _zero_acc(acc_sp, deg_sp, zrow_v, sid):
    row0 = sid * ROWS_PER_TILE
    for r in range(ROWS_PER_TILE // CH):
        pltpu.sync_copy(zrow_v, acc_sp.at[pl.ds(row0 + r * CH, CH)])
        if deg_sp is not None:
            pltpu.sync_copy(zrow_v.at[0],
                            deg_sp.at[pl.ds(row0 + r * CH, CH)])


@functools.partial(
    pl.kernel,
    out_type=(
        jax.ShapeDtypeStruct((NC, N_PAD, D), jnp.float32),
        jax.ShapeDtypeStruct((NC, N_PAD), jnp.float32),
        jax.ShapeDtypeStruct((B_OUT, D), jnp.float32),
    ),
    mesh=_mesh,
    compiler_params=_sc_params,
    scratch_types=(
        pltpu.VMEM_SHARED((N_PAD, D), jnp.float32),
        pltpu.VMEM_SHARED((N_PAD,), jnp.float32),
        pltpu.VMEM((N_NODES,), jnp.int32),
        pltpu.VMEM((STAGE,), jnp.int32),
        pltpu.VMEM((STAGE,), jnp.int32),
        pltpu.VMEM((1, CH), jnp.int32),
        pltpu.VMEM((1, CH), jnp.int32),
        pltpu.VMEM((CH, D), jnp.float32),
        pltpu.VMEM((CH,), jnp.float32),
        pltpu.VMEM((CH,), jnp.int32),
        pltpu.SemaphoreType.DMA,
    ),
)
def _sc1(emb_hbm, in_nodes_hbm, edges_hbm, onodes_hbm,
         acc_hbm, deg_hbm, origin_hbm,
         acc_sp, deg_sp, in_nodes_v, src_v, dst_v, dst2d_v, g1_v,
         rows_v, ones_v, obuf_v, sem):
    cid = lax.axis_index("c")
    sid = lax.axis_index("s")
    wid = sid * NC + cid

    pltpu.sync_copy(in_nodes_hbm, in_nodes_v)

    _zero_zrow(rows_v)
    ov = jnp.ones((L,), jnp.float32)
    for k in range(CH // L):
        ones_v[pl.ds(k * L, L)] = ov

    _zero_acc(acc_sp, deg_sp, rows_v, sid)
    plsc.subcore_barrier()

    lo = (wid * NCHUNK) // NW
    hi = ((wid + 1) * NCHUNK) // NW
    off = lo * CH
    pltpu.sync_copy(edges_hbm.at[0, pl.ds(off, STAGE)], src_v)
    pltpu.sync_copy(edges_hbm.at[1, pl.ds(off, STAGE)], dst_v)

    def edge_chunk(j, carry):
        jb = j * CH
        for k in range(CH // L):
            dst2d_v[0, pl.ds(k * L, L)] = dst_v[pl.ds(jb + k * L, L)]
            sv = src_v[pl.ds(jb + k * L, L)]
            g1_v[0, pl.ds(k * L, L)] = plsc.load_gather(in_nodes_v, [sv])
        pltpu.async_copy(emb_hbm.at[g1_v.at[0]], rows_v, sem).wait()
        pltpu.sync_copy(rows_v, acc_sp.at[dst2d_v.at[0]], add=True)
        pltpu.sync_copy(ones_v, deg_sp.at[dst2d_v.at[0]], add=True)
        return carry

    lax.fori_loop(0, hi - lo, edge_chunk, 0)
    plsc.subcore_barrier()

    row0 = sid * ROWS_PER_TILE
    for r in range(ROWS_PER_TILE // CH):
        pltpu.sync_copy(acc_sp.at[pl.ds(row0 + r * CH, CH)],
                        acc_hbm.at[cid, pl.ds(row0 + r * CH, CH)])
        pltpu.sync_copy(deg_sp.at[pl.ds(row0 + r * CH, CH)],
                        deg_hbm.at[cid, pl.ds(row0 + r * CH, CH)])

    ob = wid * CH
    pltpu.sync_copy(onodes_hbm.at[pl.ds(ob, CH)], obuf_v)
    for k in range(CH // L):
        ovk = obuf_v[pl.ds(k * L, L)]
        g1_v[0, pl.ds(k * L, L)] = plsc.load_gather(in_nodes_v, [ovk])
    pltpu.async_copy(emb_hbm.at[g1_v.at[0]], rows_v, sem).wait()
    pltpu.sync_copy(rows_v, origin_hbm.at[pl.ds(ob, CH)])


@functools.partial(
    pl.kernel,
    out_type=(
        jax.ShapeDtypeStruct((NC, B_OUT, D), jnp.float32),
        jax.ShapeDtypeStruct((B_OUT,), jnp.float32),
    ),
    mesh=_mesh,
    compiler_params=_sc_params,
    scratch_types=(
        pltpu.VMEM_SHARED((N_PAD, D), jnp.float32),
        pltpu.VMEM((STAGE,), jnp.int32),
        pltpu.VMEM((STAGE,), jnp.int32),
        pltpu.VMEM((1, CH), jnp.int32),
        pltpu.VMEM((CH, D), jnp.float32),
        pltpu.VMEM((CH,), jnp.int32),
        pltpu.VMEM((CH,), jnp.float32),
        pltpu.VMEM((CH,), jnp.float32),
        pltpu.VMEM((CH,), jnp.float32),
        pltpu.SemaphoreType.DMA,
    ),
)
def _sc2(h_hbm, edges_hbm, onodes_hbm, deg0_hbm, deg1_hbm,
         g2_hbm, degout_hbm,
         acc_sp, src_v, dst_v, dst2d_v, rows_v, obuf_v,
         dv0_v, dv1_v, dout_v, sem):
    cid = lax.axis_index("c")
    sid = lax.axis_index("s")
    wid = sid * NC + cid

    _zero_zrow(rows_v)
    _zero_acc(acc_sp, None, rows_v, sid)
    plsc.subcore_barrier()

    lo = (wid * NCHUNK) // NW
    hi = ((wid + 1) * NCHUNK) // NW
    off = lo * CH
    pltpu.sync_copy(edges_hbm.at[0, pl.ds(off, STAGE)], src_v)
    pltpu.sync_copy(edges_hbm.at[1, pl.ds(off, STAGE)], dst_v)

    def edge_chunk(j, carry):
        jb = j * CH
        for k in range(CH // L):
            dst2d_v[0, pl.ds(k * L, L)] = dst_v[pl.ds(jb + k * L, L)]
        pltpu.async_copy(h_hbm.at[src_v.at[pl.ds(jb, CH)]], rows_v, sem).wait()
        pltpu.sync_copy(rows_v, acc_sp.at[dst2d_v.at[0]], add=True)
        return carry

    lax.fori_loop(0, hi - lo, edge_chunk, 0)
    plsc.subcore_barrier()

    for j in range(B_OUT // (NS * CH)):
        o_off = (sid * 2 + j) * CH
        pltpu.sync_copy(onodes_hbm.at[pl.ds(o_off, CH)], obuf_v)
        pltpu.async_copy(acc_sp.at[obuf_v], rows_v, sem).wait()
        pltpu.sync_copy(rows_v, g2_hbm.at[cid, pl.ds(o_off, CH)])

    @pl.when(cid == 0)
    def _():
        for j in range(B_OUT // (NS * CH)):
            o_off = (sid * 2 + j) * CH
            pltpu.sync_copy(onodes_hbm.at[pl.ds(o_off, CH)], obuf_v)
            pltpu.async_copy(deg0_hbm.at[obuf_v], dv0_v, sem).wait()
            pltpu.async_copy(deg1_hbm.at[obuf_v], dv1_v, sem).wait()
            for k in range(CH // L):
                s = dv0_v[pl.ds(k * L, L)] + dv1_v[pl.ds(k * L, L)]
                dout_v[pl.ds(k * L, L)] = jnp.maximum(s, 1.0)
            pltpu.sync_copy(dout_v, degout_hbm.at[pl.ds(o_off, CH)])


BLK = 1280


def _tc1_body(acc_ref, deg_ref, w1_ref, b1_ref, h_ref):
    s = acc_ref[0] + acc_ref[1]
    d = jnp.maximum(deg_ref[0] + deg_ref[1], 1.0)
    m = s / d
    h = jnp.dot(m, w1_ref[...], preferred_element_type=jnp.float32)
    h_ref[...] = jnp.maximum(h + b1_ref[...], 0.0)


def _tc1(acc, deg3, W1, b1r):
    return pl.pallas_call(
        _tc1_body,
        grid=(N_PAD // BLK,),
        in_specs=[
            pl.BlockSpec((NC, BLK, D), lambda i: (0, i, 0)),
            pl.BlockSpec((NC, BLK, 1), lambda i: (0, i, 0)),
            pl.BlockSpec((D, D), lambda i: (0, 0)),
            pl.BlockSpec((1, D), lambda i: (0, 0)),
        ],
        out_specs=pl.BlockSpec((BLK, D), lambda i: (i, 0)),
        out_shape=jax.ShapeDtypeStruct((N_PAD, D), jnp.float32),
    )(acc, deg3, W1, b1r)


def _tc2_body(g2_ref, origin_ref, dout_ref, w2_ref, b2_ref, wc_ref, bc_ref,
              labels_ref, loss_ref):
    s2 = g2_ref[0] + g2_ref[1]
    v = s2 / dout_ref[...]
    h2 = jnp.dot(v, w2_ref[...], preferred_element_type=jnp.float32)
    h2 = jnp.maximum(h2 + b2_ref[...], 0.0)
    rep = origin_ref[...] + h2
    a = rep[:Q]
    b = rep[Q:]
    feats = jnp.concatenate([a, b, jnp.abs(a - b), a * b], axis=1)
    logits = jnp.dot(feats, wc_ref[...], preferred_element_type=jnp.float32)
    logits = logits + bc_ref[...]
    m = jnp.max(logits, axis=1, keepdims=True)
    logz = m + jnp.log(jnp.sum(jnp.exp(logits - m), axis=1, keepdims=True))
    lab = labels_ref[...]
    ll = jnp.where(lab == 0, logits[:, 0:1], logits[:, 1:2])
    loss_ref[...] = jnp.sum(logz - ll, axis=(0, 1), keepdims=True) / float(Q)


def _tc2(g2, origin, dout2, W2, b2r, Wc, bcr, labels2):
    return pl.pallas_call(
        _tc2_body,
        out_shape=jax.ShapeDtypeStruct((1, 1), jnp.float32),
    )(g2, origin, dout2, W2, b2r, Wc, bcr, labels2)


def kernel(embedding, W1, b1, W2, b2, Wc, bc, input_nodes, edge_index,
           output_nodes, labels):
    input_nodes = input_nodes.astype(jnp.int32)
    edge_index = edge_index.astype(jnp.int32)
    output_nodes = output_nodes.astype(jnp.int32)
    labels = labels.astype(jnp.int32)

    acc, deg, origin = _sc1(embedding, input_nodes, edge_index, output_nodes)
    h = _tc1(acc, deg.reshape(NC, N_PAD, 1), W1, b1.reshape(1, D))
    g2, dout = _sc2(h, edge_index, output_nodes, deg[0], deg[1])
    loss = _tc2(g2, origin, dout.reshape(B_OUT, 1), W2, b2.reshape(1, D),
                Wc, bc.reshape(1, 2), labels.reshape(Q, 1))
    return loss[0, 0]


# speedup vs baseline: 12.7367x; 1.4040x over previous
import functools

import jax
import jax.numpy as jnp
from jax import lax
from jax.experimental import pallas as pl
from jax.experimental.pallas import tpu as pltpu
from jax.experimental.pallas import tpu_sc as plsc

NC = 2
NS = 16
NW = NC * NS
L = 16

N_NODES = 10000
N_PAD = 10240
E_TOT = 320000
D = 128
CH = 128
NCHUNK = E_TOT // CH
CPW = NCHUNK // NW
NTAIL = NCHUNK - CPW * NW
NBLK = 8
B_OUT = 4096
Q = B_OUT // 2
ROWS_PER_TILE = N_PAD // NS

_mesh = plsc.VectorSubcoreMesh(core_axis_name="c", subcore_axis_name="s")
_sc_params = pltpu.CompilerParams(needs_layout_passes=False)


def _zero_rows(rows_v):
    zv = jnp.zeros((L,), jnp.float32)

    def zbody(i, carry):
        r = i // (D // L)
        k = i % (D // L)
        rows_v[r, pl.ds(k * L, L)] = zv
        return carry

    lax.fori_loop(0, CH * (D // L), zbody, 0)


def _zero_acc(acc_sp, deg_sp, zrow_v, sid):
    row0 = sid * ROWS_PER_TILE
    for r in range(ROWS_PER_TILE // CH):
        pltpu.sync_copy(zrow_v, acc_sp.at[pl.ds(row0 + r * CH, CH)])
        if deg_sp is not None:
            pltpu.sync_copy(zrow_v.at[0],
                            deg_sp.at[pl.ds(row0 + r * CH, CH)])


@functools.partial(
    pl.kernel,
    out_type=(
        jax.ShapeDtypeStruct((NC, N_PAD, D), jnp.float32),
        jax.ShapeDtypeStruct((NC, N_PAD), jnp.float32),
        jax.ShapeDtypeStruct((B_OUT, D), jnp.float32),
    ),
    mesh=_mesh,
    compiler_params=_sc_params,
    scratch_types=(
        pltpu.VMEM_SHARED((N_PAD, D), jnp.float32),
        pltpu.VMEM_SHARED((N_PAD,), jnp.float32),
        pltpu.VMEM((N_NODES,), jnp.int32),
        pltpu.VMEM((NBLK * CH,), jnp.int32),
        pltpu.VMEM((NBLK * CH,), jnp.int32),
        pltpu.VMEM((1, CH), jnp.int32),
        pltpu.VMEM((1, CH), jnp.int32),
        pltpu.VMEM((1, CH), jnp.int32),
        pltpu.VMEM((1, CH), jnp.int32),
        pltpu.VMEM((CH, D), jnp.float32),
        pltpu.VMEM((CH, D), jnp.float32),
        pltpu.VMEM((CH,), jnp.float32),
        pltpu.VMEM((CH,), jnp.int32),
        pltpu.SemaphoreType.DMA,
        pltpu.SemaphoreType.DMA,
    ),
)
def _sc1(emb_hbm, in_nodes_hbm, edges_hbm, onodes_hbm,
         acc_hbm, deg_hbm, origin_hbm,
         acc_sp, deg_sp, in_nodes_v, src_blk, dst_blk,
         g1c0, g1c1, dst2d0, dst2d1, rows0, rows1, ones_v, obuf_v,
         semg0, semg1):
    cid = lax.axis_index("c")
    sid = lax.axis_index("s")
    wid = sid * NC + cid
    base = wid * CPW

    g1c = (g1c0, g1c1)
    dst2d = (dst2d0, dst2d1)
    rows = (rows0, rows1)
    semg = (semg0, semg1)

    pltpu.sync_copy(in_nodes_hbm, in_nodes_v)

    _zero_rows(rows0)
    ov = jnp.ones((L,), jnp.float32)
    for k in range(CH // L):
        ones_v[pl.ds(k * L, L)] = ov

    _zero_acc(acc_sp, deg_sp, rows0, sid)
    plsc.subcore_barrier()

    def refill(j):
        @pl.when(lax.rem(j, NBLK) == 0)
        def _():
            off = (base + j) * CH
            pltpu.sync_copy(edges_hbm.at[0, pl.ds(off, NBLK * CH)], src_blk)
            pltpu.sync_copy(edges_hbm.at[1, pl.ds(off, NBLK * CH)], dst_blk)

    def launch(j, b):
        refill(j)
        jo = lax.rem(j, NBLK) * CH
        for k in range(CH // L):
            sv = src_blk[pl.ds(jo + k * L, L)]
            g1c[b][0, pl.ds(k * L, L)] = plsc.load_gather(in_nodes_v, [sv])
            dst2d[b][0, pl.ds(k * L, L)] = dst_blk[pl.ds(jo + k * L, L)]
        pltpu.async_copy(emb_hbm.at[g1c[b].at[0]], rows[b], semg[b])

    def finish(b):
        pltpu.make_async_copy(emb_hbm.at[g1c[b].at[0]], rows[b],
                              semg[b]).wait()
        pltpu.sync_copy(rows[b], acc_sp.at[dst2d[b].at[0]], add=True)
        pltpu.sync_copy(ones_v, deg_sp.at[dst2d[b].at[0]], add=True)

    launch(0, 0)

    def pair(g, carry):
        j0 = 2 * g
        launch(j0 + 1, 1)
        finish(0)

        @pl.when(j0 + 2 < CPW)
        def _():
            launch(j0 + 2, 0)

        finish(1)
        return carry

    lax.fori_loop(0, CPW // 2, pair, 0)

    @pl.when(wid < NTAIL)
    def _():
        c = (NW * CPW + wid) * CH
        pltpu.sync_copy(edges_hbm.at[0, pl.ds(c, CH)],
                        src_blk.at[pl.ds(0, CH)])
        pltpu.sync_copy(edges_hbm.at[1, pl.ds(c, CH)],
                        dst_blk.at[pl.ds(0, CH)])
        for k in range(CH // L):
            sv = src_blk[pl.ds(k * L, L)]
            g1c0[0, pl.ds(k * L, L)] = plsc.load_gather(in_nodes_v, [sv])
            dst2d0[0, pl.ds(k * L, L)] = dst_blk[pl.ds(k * L, L)]
        pltpu.async_copy(emb_hbm.at[g1c0.at[0]], rows0, semg0).wait()
        pltpu.sync_copy(rows0, acc_sp.at[dst2d0.at[0]], add=True)
        pltpu.sync_copy(ones_v, deg_sp.at[dst2d0.at[0]], add=True)

    plsc.subcore_barrier()

    row0 = sid * ROWS_PER_TILE
    for r in range(ROWS_PER_TILE // CH):
        pltpu.sync_copy(acc_sp.at[pl.ds(row0 + r * CH, CH)],
                        acc_hbm.at[cid, pl.ds(row0 + r * CH, CH)])
        pltpu.sync_copy(deg_sp.at[pl.ds(row0 + r * CH, CH)],
                        deg_hbm.at[cid, pl.ds(row0 + r * CH, CH)])

    ob = wid * CH
    pltpu.sync_copy(onodes_hbm.at[pl.ds(ob, CH)], obuf_v)
    for k in range(CH // L):
        ovk = obuf_v[pl.ds(k * L, L)]
        g1c0[0, pl.ds(k * L, L)] = plsc.load_gather(in_nodes_v, [ovk])
    pltpu.async_copy(emb_hbm.at[g1c0.at[0]], rows0, semg0).wait()
    pltpu.sync_copy(rows0, origin_hbm.at[pl.ds(ob, CH)])


@functools.partial(
    pl.kernel,
    out_type=(
        jax.ShapeDtypeStruct((NC, B_OUT, D), jnp.float32),
        jax.ShapeDtypeStruct((B_OUT,), jnp.float32),
    ),
    mesh=_mesh,
    compiler_params=_sc_params,
    scratch_types=(
        pltpu.VMEM_SHARED((N_PAD, D), jnp.float32),
        pltpu.VMEM((NBLK * CH,), jnp.int32),
        pltpu.VMEM((NBLK * CH,), jnp.int32),
        pltpu.VMEM((1, CH), jnp.int32),
        pltpu.VMEM((1, CH), jnp.int32),
        pltpu.VMEM((1, CH), jnp.int32),
        pltpu.VMEM((1, CH), jnp.int32),
        pltpu.VMEM((CH, D), jnp.float32),
        pltpu.VMEM((CH, D), jnp.float32),
        pltpu.VMEM((CH,), jnp.int32),
        pltpu.VMEM((CH,), jnp.float32),
        pltpu.VMEM((CH,), jnp.float32),
        pltpu.VMEM((CH,), jnp.float32),
        pltpu.SemaphoreType.DMA,
        pltpu.SemaphoreType.DMA,
    ),
)
def _sc2(h_hbm, edges_hbm, onodes_hbm, deg0_hbm, deg1_hbm,
         g2_hbm, degout_hbm,
         acc_sp, src_blk, dst_blk, srcc0, srcc1, dst2d0, dst2d1,
         rows0, rows1, obuf_v, dv0_v, dv1_v, dout_v, semg0, semg1):
    cid = lax.axis_index("c")
    sid = lax.axis_index("s")
    wid = sid * NC + cid
    base = wid * CPW

    srcc = (srcc0, srcc1)
    dst2d = (dst2d0, dst2d1)
    rows = (rows0, rows1)
    semg = (semg0, semg1)

    _zero_rows(rows0)
    _zero_acc(acc_sp, None, rows0, sid)
    plsc.subcore_barrier()

    def refill(j):
        @pl.when(lax.rem(j, NBLK) == 0)
        def _():
            off = (base + j) * CH
            pltpu.sync_copy(edges_hbm.at[0, pl.ds(off, NBLK * CH)], src_blk)
            pltpu.sync_copy(edges_hbm.at[1, pl.ds(off, NBLK * CH)], dst_blk)

    def launch(j, b):
        refill(j)
        jo = lax.rem(j, NBLK) * CH
        for k in range(CH // L):
            srcc[b][0, pl.ds(k * L, L)] = src_blk[pl.ds(jo + k * L, L)]
            dst2d[b][0, pl.ds(k * L, L)] = dst_blk[pl.ds(jo + k * L, L)]
        pltpu.async_copy(h_hbm.at[srcc[b].at[0]], rows[b], semg[b])

    def finish(b):
        pltpu.make_async_copy(h_hbm.at[srcc[b].at[0]], rows[b],
                              semg[b]).wait()
        pltpu.sync_copy(rows[b], acc_sp.at[dst2d[b].at[0]], add=True)

    launch(0, 0)

    def pair(g, carry):
        j0 = 2 * g
        launch(j0 + 1, 1)
        finish(0)

        @pl.when(j0 + 2 < CPW)
        def _():
            launch(j0 + 2, 0)

        finish(1)
        return carry

    lax.fori_loop(0, CPW // 2, pair, 0)

    @pl.when(wid < NTAIL)
    def _():
        c = (NW * CPW + wid) * CH
        pltpu.sync_copy(edges_hbm.at[0, pl.ds(c, CH)],
                        src_blk.at[pl.ds(0, CH)])
        pltpu.sync_copy(edges_hbm.at[1, pl.ds(c, CH)],
                        dst_blk.at[pl.ds(0, CH)])
        for k in range(CH // L):
            srcc0[0, pl.ds(k * L, L)] = src_blk[pl.ds(k * L, L)]
            dst2d0[0, pl.ds(k * L, L)] = dst_blk[pl.ds(k * L, L)]
        pltpu.async_copy(h_hbm.at[srcc0.at[0]], rows0, semg0).wait()
        pltpu.sync_copy(rows0, acc_sp.at[dst2d0.at[0]], add=True)

    plsc.subcore_barrier()

    for j in range(B_OUT // (NS * CH)):
        o_off = (sid * 2 + j) * CH
        pltpu.sync_copy(onodes_hbm.at[pl.ds(o_off, CH)], obuf_v)
        pltpu.async_copy(acc_sp.at[obuf_v], rows0, semg0).wait()
        pltpu.sync_copy(rows0, g2_hbm.at[cid, pl.ds(o_off, CH)])

    @pl.when(cid == 0)
    def _():
        for j in range(B_OUT // (NS * CH)):
            o_off = (sid * 2 + j) * CH
            pltpu.sync_copy(onodes_hbm.at[pl.ds(o_off, CH)], obuf_v)
            pltpu.async_copy(deg0_hbm.at[obuf_v], dv0_v, semg0).wait()
            pltpu.async_copy(deg1_hbm.at[obuf_v], dv1_v, semg0).wait()
            for k in range(CH // L):
                s = dv0_v[pl.ds(k * L, L)] + dv1_v[pl.ds(k * L, L)]
                dout_v[pl.ds(k * L, L)] = jnp.maximum(s, 1.0)
            pltpu.sync_copy(dout_v, degout_hbm.at[pl.ds(o_off, CH)])


BLK = 1280


def _tc1_body(acc_ref, deg_ref, w1_ref, b1_ref, h_ref):
    s = acc_ref[0] + acc_ref[1]
    d = jnp.maximum(deg_ref[0] + deg_ref[1], 1.0)
    m = s / d
    h = jnp.dot(m, w1_ref[...], preferred_element_type=jnp.float32)
    h_ref[...] = jnp.maximum(h + b1_ref[...], 0.0)


def _tc1(acc, deg3, W1, b1r):
    return pl.pallas_call(
        _tc1_body,
        grid=(N_PAD // BLK,),
        in_specs=[
            pl.BlockSpec((NC, BLK, D), lambda i: (0, i, 0)),
            pl.BlockSpec((NC, BLK, 1), lambda i: (0, i, 0)),
            pl.BlockSpec((D, D), lambda i: (0, 0)),
            pl.BlockSpec((1, D), lambda i: (0, 0)),
        ],
        out_specs=pl.BlockSpec((BLK, D), lambda i: (i, 0)),
        out_shape=jax.ShapeDtypeStruct((N_PAD, D), jnp.float32),
    )(acc, deg3, W1, b1r)


def _tc2_body(g2_ref, origin_ref, dout_ref, w2_ref, b2_ref, wc_ref, bc_ref,
              labels_ref, loss_ref):
    s2 = g2_ref[0] + g2_ref[1]
    v = s2 / dout_ref[...]
    h2 = jnp.dot(v, w2_ref[...], preferred_element_type=jnp.float32)
    h2 = jnp.maximum(h2 + b2_ref[...], 0.0)
    rep = origin_ref[...] + h2
    a = rep[:Q]
    b = rep[Q:]
    feats = jnp.concatenate([a, b, jnp.abs(a - b), a * b], axis=1)
    logits = jnp.dot(feats, wc_ref[...], preferred_element_type=jnp.float32)
    logits = logits + bc_ref[...]
    m = jnp.max(logits, axis=1, keepdims=True)
    logz = m + jnp.log(jnp.sum(jnp.exp(logits - m), axis=1, keepdims=True))
    lab = labels_ref[...]
    ll = jnp.where(lab == 0, logits[:, 0:1], logits[:, 1:2])
    loss_ref[...] = jnp.sum(logz - ll, axis=(0, 1), keepdims=True) / float(Q)


def _tc2(g2, origin, dout2, W2, b2r, Wc, bcr, labels2):
    return pl.pallas_call(
        _tc2_body,
        out_shape=jax.ShapeDtypeStruct((1, 1), jnp.float32),
    )(g2, origin, dout2, W2, b2r, Wc, bcr, labels2)


def kernel(embedding, W1, b1, W2, b2, Wc, bc, input_nodes, edge_index,
           output_nodes, labels):
    input_nodes = input_nodes.astype(jnp.int32)
    edge_index = edge_index.astype(jnp.int32)
    output_nodes = output_nodes.astype(jnp.int32)
    labels = labels.astype(jnp.int32)

    acc, deg, origin = _sc1(embedding, input_nodes, edge_index, output_nodes)
    h = _tc1(acc, deg.reshape(NC, N_PAD, 1), W1, b1.reshape(1, D))
    g2, dout = _sc2(h, edge_index, output_nodes, deg[0], deg[1])
    loss = _tc2(g2, origin, dout.reshape(B_OUT, 1), W2, b2.reshape(1, D),
                Wc, bc.reshape(1, 2), labels.reshape(Q, 1))
    return loss[0, 0]


# speedup vs baseline: 13.1010x; 1.0286x over previous
import functools

import jax
import jax.numpy as jnp
from jax import lax
from jax.experimental import pallas as pl
from jax.experimental.pallas import tpu as pltpu
from jax.experimental.pallas import tpu_sc as plsc

NC = 2
NS = 16
NW = NC * NS
L = 16

N_NODES = 10000
N_PAD = 10240
E_TOT = 320000
D = 128
CH = 80
CPW = 128
NCHUNK = CPW * NW
E_PAD = NCHUNK * CH
NBLK = 16
NB = 3
B_OUT = 4096
Q = B_OUT // 2
OCH = 64
ROWS_PER_TILE = N_PAD // NS

_mesh = plsc.VectorSubcoreMesh(core_axis_name="c", subcore_axis_name="s")
_sc_params = pltpu.CompilerParams(needs_layout_passes=False)


def _zero_buf(rows_v):
    zv = jnp.zeros((L,), jnp.float32)

    def zbody(i, carry):
        r = i // (D // L)
        k = i % (D // L)
        rows_v[r, pl.ds(k * L, L)] = zv
        return carry

    lax.fori_loop(0, CH * (D // L), zbody, 0)


def _zero_acc(acc_sp, deg_sp, zrow_v, sid):
    row0 = sid * ROWS_PER_TILE
    for r in range(ROWS_PER_TILE // CH):
        pltpu.sync_copy(zrow_v, acc_sp.at[pl.ds(row0 + r * CH, CH)])
        if deg_sp is not None:
            pltpu.sync_copy(zrow_v.at[0, pl.ds(0, CH)],
                            deg_sp.at[pl.ds(row0 + r * CH, CH)])


@functools.partial(
    pl.kernel,
    out_type=(
        jax.ShapeDtypeStruct((NC, N_PAD, D), jnp.float32),
        jax.ShapeDtypeStruct((NC * N_PAD,), jnp.float32),
        jax.ShapeDtypeStruct((B_OUT, D), jnp.float32),
    ),
    mesh=_mesh,
    compiler_params=_sc_params,
    scratch_types=(
        pltpu.VMEM_SHARED((N_PAD, D), jnp.float32),
        pltpu.VMEM_SHARED((N_PAD,), jnp.float32),
        pltpu.VMEM((N_NODES,), jnp.int32),
        pltpu.VMEM((2, NBLK * CH), jnp.int32),
        pltpu.VMEM((1, CH), jnp.int32),
        pltpu.VMEM((1, CH), jnp.int32),
        pltpu.VMEM((1, CH), jnp.int32),
        pltpu.VMEM((1, CH), jnp.int32),
        pltpu.VMEM((1, CH), jnp.int32),
        pltpu.VMEM((1, CH), jnp.int32),
        pltpu.VMEM((CH, D), jnp.float32),
        pltpu.VMEM((CH, D), jnp.float32),
        pltpu.VMEM((CH, D), jnp.float32),
        pltpu.VMEM((CH,), jnp.float32),
        pltpu.VMEM((OCH,), jnp.int32),
        pltpu.SemaphoreType.DMA,
        pltpu.SemaphoreType.DMA,
        pltpu.SemaphoreType.DMA,
        pltpu.SemaphoreType.DMA,
        pltpu.SemaphoreType.DMA,
        pltpu.SemaphoreType.DMA,
        pltpu.SemaphoreType.DMA,
        pltpu.SemaphoreType.DMA,
        pltpu.SemaphoreType.DMA,
    ),
)
def _sc1(emb_hbm, in_nodes_hbm, edges_hbm, onodes_hbm,
         acc_hbm, deg_hbm, origin_hbm,
         acc_sp, deg_sp, in_nodes_v, stage,
         g1c0, g1c1, g1c2, dst2d0, dst2d1, dst2d2, rows0, rows1, rows2,
         ones_v, obuf_v,
         semg0, semg1, semg2, sems0, sems1, sems2, semd0, semd1, semd2):
    cid = lax.axis_index("c")
    sid = lax.axis_index("s")
    wid = sid * NC + cid
    base = wid * CPW

    g1c = (g1c0, g1c1, g1c2)
    dst2d = (dst2d0, dst2d1, dst2d2)
    rows = (rows0, rows1, rows2)
    semg = (semg0, semg1, semg2)
    sems = (sems0, sems1, sems2)
    semd = (semd0, semd1, semd2)

    pltpu.sync_copy(in_nodes_hbm, in_nodes_v)

    _zero_buf(rows0)
    ov = jnp.ones((L,), jnp.float32)
    for k in range(CH // L):
        ones_v[pl.ds(k * L, L)] = ov

    _zero_acc(acc_sp, deg_sp, rows0, sid)
    plsc.subcore_barrier()

    def launch(j, b):
        @pl.when(j >= NB)
        def _():
            pltpu.make_async_copy(rows[b], acc_sp.at[dst2d[b].at[0]],
                                  sems[b]).wait()
            pltpu.make_async_copy(ones_v, deg_sp.at[dst2d[b].at[0]],
                                  semd[b]).wait()

        @pl.when(lax.rem(j, NBLK) == 0)
        def _():
            off = pl.multiple_of((base + j) * CH, NBLK * CH)
            pltpu.sync_copy(edges_hbm.at[pl.ds(0, 2), pl.ds(off, NBLK * CH)],
                            stage)

        jo = lax.rem(j, NBLK) * CH
        for k in range(CH // L):
            sv = stage[0, pl.ds(jo + k * L, L)]
            g1c[b][0, pl.ds(k * L, L)] = plsc.load_gather(in_nodes_v, [sv])
            dst2d[b][0, pl.ds(k * L, L)] = stage[1, pl.ds(jo + k * L, L)]
        pltpu.async_copy(emb_hbm.at[g1c[b].at[0]], rows[b], semg[b])

    def step(j, b):
        pltpu.make_async_copy(emb_hbm.at[g1c[b].at[0]], rows[b],
                              semg[b]).wait()
        pltpu.async_copy(rows[b], acc_sp.at[dst2d[b].at[0]], sems[b],
                         add=True)
        pltpu.async_copy(ones_v, deg_sp.at[dst2d[b].at[0]], semd[b],
                         add=True)

    launch(0, 0)
    launch(1, 1)

    def tri(g, carry):
        j0 = 3 * g
        for k in range(NB):
            step(j0 + k, k)
            launch(j0 + k + 2, (k + 2) % NB)
        return carry

    lax.fori_loop(0, (CPW - 2) // NB, tri, 0)
    step(CPW - 2, (CPW - 2) % NB)
    step(CPW - 1, (CPW - 1) % NB)

    for b in range(NB):
        pltpu.make_async_copy(rows[b], acc_sp.at[dst2d[b].at[0]],
                              sems[b]).wait()
        pltpu.make_async_copy(ones_v, deg_sp.at[dst2d[b].at[0]],
                              semd[b]).wait()

    plsc.subcore_barrier()

    row0 = sid * ROWS_PER_TILE
    for r in range(ROWS_PER_TILE // CH):
        pltpu.sync_copy(acc_sp.at[pl.ds(row0 + r * CH, CH)],
                        acc_hbm.at[cid, pl.ds(row0 + r * CH, CH)])
    for r in range(ROWS_PER_TILE // D):
        pltpu.sync_copy(deg_sp.at[pl.ds(row0 + r * D, D)],
                        deg_hbm.at[pl.ds(cid * N_PAD + row0 + r * D, D)])

    for q in range(2):
        ob = wid * 2 * OCH + q * OCH
        pltpu.sync_copy(onodes_hbm.at[pl.ds(ob, OCH)], obuf_v)
        for k in range(OCH // L):
            ovk = obuf_v[pl.ds(k * L, L)]
            g1c0[0, pl.ds(k * L, L)] = plsc.load_gather(in_nodes_v, [ovk])
        pltpu.async_copy(emb_hbm.at[g1c0.at[0, pl.ds(0, OCH)]],
                         rows0.at[pl.ds(0, OCH)], semg0).wait()
        pltpu.sync_copy(rows0.at[pl.ds(0, OCH)],
                        origin_hbm.at[pl.ds(ob, OCH)])


@functools.partial(
    pl.kernel,
    out_type=(
        jax.ShapeDtypeStruct((NC, B_OUT, D), jnp.float32),
        jax.ShapeDtypeStruct((B_OUT,), jnp.float32),
    ),
    mesh=_mesh,
    compiler_params=_sc_params,
    scratch_types=(
        pltpu.VMEM_SHARED((N_PAD, D), jnp.float32),
        pltpu.VMEM((2, NBLK * CH), jnp.int32),
        pltpu.VMEM((1, CH), jnp.int32),
        pltpu.VMEM((1, CH), jnp.int32),
        pltpu.VMEM((1, CH), jnp.int32),
        pltpu.VMEM((1, CH), jnp.int32),
        pltpu.VMEM((1, CH), jnp.int32),
        pltpu.VMEM((1, CH), jnp.int32),
        pltpu.VMEM((CH, D), jnp.float32),
        pltpu.VMEM((CH, D), jnp.float32),
        pltpu.VMEM((CH, D), jnp.float32),
        pltpu.VMEM((OCH,), jnp.int32),
        pltpu.VMEM((OCH,), jnp.float32),
        pltpu.VMEM((OCH,), jnp.float32),
        pltpu.VMEM((OCH,), jnp.float32),
        pltpu.SemaphoreType.DMA,
        pltpu.SemaphoreType.DMA,
        pltpu.SemaphoreType.DMA,
        pltpu.SemaphoreType.DMA,
        pltpu.SemaphoreType.DMA,
        pltpu.SemaphoreType.DMA,
    ),
)
def _sc2(h_hbm, edges_hbm, onodes_hbm, deg0_hbm, deg1_hbm,
         g2_hbm, degout_hbm,
         acc_sp, stage, srcc0, srcc1, srcc2, dst2d0, dst2d1, dst2d2,
         rows0, rows1, rows2, obuf_v, dv0_v, dv1_v, dout_v,
         semg0, semg1, semg2, sems0, sems1, sems2):
    cid = lax.axis_index("c")
    sid = lax.axis_index("s")
    wid = sid * NC + cid
    base = wid * CPW

    srcc = (srcc0, srcc1, srcc2)
    dst2d = (dst2d0, dst2d1, dst2d2)
    rows = (rows0, rows1, rows2)
    semg = (semg0, semg1, semg2)
    sems = (sems0, sems1, sems2)

    _zero_buf(rows0)
    _zero_acc(acc_sp, None, rows0, sid)
    plsc.subcore_barrier()

    def launch(j, b):
        @pl.when(j >= NB)
        def _():
            pltpu.make_async_copy(rows[b], acc_sp.at[dst2d[b].at[0]],
                                  sems[b]).wait()

        @pl.when(lax.rem(j, NBLK) == 0)
        def _():
            off = pl.multiple_of((base + j) * CH, NBLK * CH)
            pltpu.sync_copy(edges_hbm.at[pl.ds(0, 2), pl.ds(off, NBLK * CH)],
                            stage)

        jo = lax.rem(j, NBLK) * CH
        for k in range(CH // L):
            srcc[b][0, pl.ds(k * L, L)] = stage[0, pl.ds(jo + k * L, L)]
            dst2d[b][0, pl.ds(k * L, L)] = stage[1, pl.ds(jo + k * L, L)]
        pltpu.async_copy(h_hbm.at[srcc[b].at[0]], rows[b], semg[b])

    def step(j, b):
        pltpu.make_async_copy(h_hbm.at[srcc[b].at[0]], rows[b],
                              semg[b]).wait()
        pltpu.async_copy(rows[b], acc_sp.at[dst2d[b].at[0]], sems[b],
                         add=True)

    launch(0, 0)
    launch(1, 1)

    def tri(g, carry):
        j0 = 3 * g
        for k in range(NB):
            step(j0 + k, k)
            launch(j0 + k + 2, (k + 2) % NB)
        return carry

    lax.fori_loop(0, (CPW - 2) // NB, tri, 0)
    step(CPW - 2, (CPW - 2) % NB)
    step(CPW - 1, (CPW - 1) % NB)

    for b in range(NB):
        pltpu.make_async_copy(rows[b], acc_sp.at[dst2d[b].at[0]],
                              sems[b]).wait()

    plsc.subcore_barrier()

    for j in range(B_OUT // (NS * OCH)):
        o_off = (sid * 4 + j) * OCH
        pltpu.sync_copy(onodes_hbm.at[pl.ds(o_off, OCH)], obuf_v)
        pltpu.async_copy(acc_sp.at[obuf_v], rows0.at[pl.ds(0, OCH)],
                         semg0).wait()
        pltpu.sync_copy(rows0.at[pl.ds(0, OCH)],
                        g2_hbm.at[cid, pl.ds(o_off, OCH)])

    @pl.when(cid == 0)
    def _():
        for j in range(B_OUT // (NS * OCH)):
            o_off = (sid * 4 + j) * OCH
            pltpu.sync_copy(onodes_hbm.at[pl.ds(o_off, OCH)], obuf_v)
            pltpu.async_copy(deg0_hbm.at[obuf_v], dv0_v, semg0).wait()
            pltpu.async_copy(deg1_hbm.at[obuf_v], dv1_v, semg0).wait()
            for k in range(OCH // L):
                s = dv0_v[pl.ds(k * L, L)] + dv1_v[pl.ds(k * L, L)]
                dout_v[pl.ds(k * L, L)] = jnp.maximum(s, 1.0)
            pltpu.sync_copy(dout_v, degout_hbm.at[pl.ds(o_off, OCH)])


BLK = 1280


def _tc1_body(acc_ref, deg_ref, w1_ref, b1_ref, h_ref):
    s = acc_ref[0] + acc_ref[1]
    d = jnp.maximum(deg_ref[0] + deg_ref[1], 1.0)
    m = s / d
    h = jnp.dot(m, w1_ref[...], preferred_element_type=jnp.float32)
    h_ref[...] = jnp.maximum(h + b1_ref[...], 0.0)


def _tc1(acc, deg3, W1, b1r):
    return pl.pallas_call(
        _tc1_body,
        grid=(N_PAD // BLK,),
        in_specs=[
            pl.BlockSpec((NC, BLK, D), lambda i: (0, i, 0)),
            pl.BlockSpec((NC, BLK, 1), lambda i: (0, i, 0)),
            pl.BlockSpec((D, D), lambda i: (0, 0)),
            pl.BlockSpec((1, D), lambda i: (0, 0)),
        ],
        out_specs=pl.BlockSpec((BLK, D), lambda i: (i, 0)),
        out_shape=jax.ShapeDtypeStruct((N_PAD, D), jnp.float32),
    )(acc, deg3, W1, b1r)


def _tc2_body(g2_ref, origin_ref, dout_ref, w2_ref, b2_ref, wc_ref, bc_ref,
              labels_ref, loss_ref):
    s2 = g2_ref[0] + g2_ref[1]
    v = s2 / dout_ref[...]
    h2 = jnp.dot(v, w2_ref[...], preferred_element_type=jnp.float32)
    h2 = jnp.maximum(h2 + b2_ref[...], 0.0)
    rep = origin_ref[...] + h2
    a = rep[:Q]
    b = rep[Q:]
    feats = jnp.concatenate([a, b, jnp.abs(a - b), a * b], axis=1)
    logits = jnp.dot(feats, wc_ref[...], preferred_element_type=jnp.float32)
    logits = logits + bc_ref[...]
    m = jnp.max(logits, axis=1, keepdims=True)
    logz = m + jnp.log(jnp.sum(jnp.exp(logits - m), axis=1, keepdims=True))
    lab = labels_ref[...]
    ll = jnp.where(lab == 0, logits[:, 0:1], logits[:, 1:2])
    loss_ref[...] = jnp.sum(logz - ll, axis=(0, 1), keepdims=True) / float(Q)


def _tc2(g2, origin, dout2, W2, b2r, Wc, bcr, labels2):
    return pl.pallas_call(
        _tc2_body,
        out_shape=jax.ShapeDtypeStruct((1, 1), jnp.float32),
    )(g2, origin, dout2, W2, b2r, Wc, bcr, labels2)


def kernel(embedding, W1, b1, W2, b2, Wc, bc, input_nodes, edge_index,
           output_nodes, labels):
    input_nodes = input_nodes.astype(jnp.int32)
    edge_index = edge_index.astype(jnp.int32)
    output_nodes = output_nodes.astype(jnp.int32)
    labels = labels.astype(jnp.int32)

    npad_e = E_PAD - E_TOT
    pad_src = (jnp.arange(npad_e, dtype=jnp.int32) * 13) % N_NODES
    pad_dst = N_NODES + jnp.arange(npad_e, dtype=jnp.int32) % (N_PAD - N_NODES)
    edges_p = jnp.concatenate(
        [edge_index, jnp.stack([pad_src, pad_dst])], axis=1)

    acc, degf, origin = _sc1(embedding, input_nodes, edges_p, output_nodes)
    deg = degf.reshape(NC, N_PAD)
    h = _tc1(acc, deg.reshape(NC, N_PAD, 1), W1, b1.reshape(1, D))
    g2, dout = _sc2(h, edges_p, output_nodes, deg[0], deg[1])
    loss = _tc2(g2, origin, dout.reshape(B_OUT, 1), W2, b2.reshape(1, D),
                Wc, bc.reshape(1, 2), labels.reshape(Q, 1))
    return loss[0, 0]
